# baseline (speedup 1.0000x reference)
import functools

import jax
import jax.numpy as jnp
from jax import lax
from jax.experimental import pallas as pl
from jax.experimental.pallas import tpu as pltpu

MASK_RATIO = 0.4
MASK_VALUE = 0.0


def _mask_fill_kernel(noise_ref, patch_ref, out_ref, mask_ref, *, num_remove):
    bb, c, s = noise_ref.shape
    n = noise_ref[...].reshape(bb * c, s)
    R, S = n.shape
    k2 = pltpu.bitcast(n, jnp.int32) * 2
    j_idx = lax.broadcasted_iota(jnp.int32, (1, S, S), 1)
    i_idx = lax.broadcasted_iota(jnp.int32, (1, S, S), 2)
    tri = (j_idx > i_idx).astype(jnp.int32)
    bj = k2[:, :, None] + tri
    greater = bj > k2[:, None, :]
    cnt = jnp.count_nonzero(greater, axis=1).astype(jnp.int32)
    remove = cnt < num_remove
    mask_ref[...] = remove.astype(jnp.float32).reshape(bb, c, s)
    f = patch_ref.shape[-1]
    x = patch_ref[...].reshape(R, S, f)
    filled = jnp.where(cnt[:, :, None] < num_remove, jnp.float32(MASK_VALUE), x)
    out_ref[...] = filled.reshape(bb, c, s, f)


def kernel(patch_input, noise):
    batch, channels, seq, feat = patch_input.shape
    len_keep = int(seq * (1 - MASK_RATIO))
    num_remove = seq - len_keep

    bb = 2
    grid = (batch // bb,)

    out, mask = pl.pallas_call(
        functools.partial(_mask_fill_kernel, num_remove=num_remove),
        grid=grid,
        in_specs=[
            pl.BlockSpec((bb, channels, seq), lambda b: (b, 0, 0)),
            pl.BlockSpec((bb, channels, seq, feat), lambda b: (b, 0, 0, 0)),
        ],
        out_specs=[
            pl.BlockSpec((bb, channels, seq, feat), lambda b: (b, 0, 0, 0)),
            pl.BlockSpec((bb, channels, seq), lambda b: (b, 0, 0)),
        ],
        out_shape=[
            jax.ShapeDtypeStruct((batch, channels, seq, feat), patch_input.dtype),
            jax.ShapeDtypeStruct((batch, channels, seq), jnp.float32),
        ],
    )(noise, patch_input)

    return out, mask.astype(bool)


# speedup vs baseline: 1.0458x; 1.0458x over previous
import functools

import jax
import jax.numpy as jnp
from jax import lax
from jax.experimental import pallas as pl
from jax.experimental.pallas import tpu as pltpu

MASK_RATIO = 0.4
MASK_VALUE = 0.0

NBUF = 8


def _rank_mask(n, num_remove):
    R, S = n.shape
    k2 = pltpu.bitcast(n, jnp.int32) * 2
    j_idx = lax.broadcasted_iota(jnp.int32, (1, S, S), 1)
    i_idx = lax.broadcasted_iota(jnp.int32, (1, S, S), 2)
    tri = (j_idx > i_idx).astype(jnp.int32)
    bj = k2[:, :, None] + tri
    greater = bj > k2[:, None, :]
    cnt = jnp.count_nonzero(greater, axis=1).astype(jnp.int32)
    return cnt


def _masking_kernel(noise_ref, patch_hbm, out_hbm, mask_ref,
                    inbuf, obuf, insem, osem, *, num_remove, n_chunks):
    def in_copy(chunk, slot):
        return pltpu.make_async_copy(
            patch_hbm.at[pl.ds(chunk, 1)], inbuf.at[slot], insem.at[slot])

    def out_copy(chunk, slot):
        return pltpu.make_async_copy(
            obuf.at[slot], out_hbm.at[pl.ds(chunk, 1)], osem.at[slot])

    for s in range(NBUF):
        in_copy(s, s).start()

    def body(c, carry):
        slot = lax.rem(c, NBUF)
        in_copy(c, slot).wait()
        n = noise_ref[c]
        cnt = _rank_mask(n, num_remove)
        mask_ref[c] = (cnt < num_remove).astype(jnp.float32)
        x = inbuf[slot, 0]

        @pl.when(c >= NBUF)
        def _wait_prev_out():
            out_copy(c - NBUF, slot).wait()

        obuf[slot, 0] = jnp.where(cnt[:, :, None] < num_remove,
                                  jnp.float32(MASK_VALUE), x)
        out_copy(c, slot).start()

        @pl.when(c + NBUF < n_chunks)
        def _start_next_in():
            in_copy(c + NBUF, slot).start()

        return carry

    lax.fori_loop(0, n_chunks, body, 0)

    for s in range(NBUF):
        chunk = n_chunks - NBUF + s
        out_copy(chunk, chunk % NBUF).wait()


def kernel(patch_input, noise):
    batch, channels, seq, feat = patch_input.shape
    len_keep = int(seq * (1 - MASK_RATIO))
    num_remove = seq - len_keep

    out, mask = pl.pallas_call(
        functools.partial(_masking_kernel, num_remove=num_remove,
                          n_chunks=batch),
        in_specs=[
            pl.BlockSpec(memory_space=pltpu.VMEM),
            pl.BlockSpec(memory_space=pl.ANY),
        ],
        out_specs=[
            pl.BlockSpec(memory_space=pl.ANY),
            pl.BlockSpec(memory_space=pltpu.VMEM),
        ],
        out_shape=[
            jax.ShapeDtypeStruct((batch, channels, seq, feat), patch_input.dtype),
            jax.ShapeDtypeStruct((batch, channels, seq), jnp.float32),
        ],
        scratch_shapes=[
            pltpu.VMEM((NBUF, 1, channels, seq, feat), jnp.float32),
            pltpu.VMEM((NBUF, 1, channels, seq, feat), jnp.float32),
            pltpu.SemaphoreType.DMA((NBUF,)),
            pltpu.SemaphoreType.DMA((NBUF,)),
        ],
    )(noise, patch_input)

    return out, mask.astype(bool)
